# baseline (speedup 1.0000x reference)
import jax
import jax.numpy as jnp
from jax import lax
from jax.experimental import pallas as pl
from jax.experimental.pallas import tpu as pltpu
from jax.experimental.pallas import tpu_sc as plsc

M = 524288
D = 64
B = 65536

NC = 2
NS = 16
NPASS = 16
RANGE = M // (NPASS * NC)
TROWS = RANGE // NS
SLICE = B // NS
CHUNKS = SLICE // 16
GARBAGE = RANGE
DUMP = SLICE


def _body(mem_hbm, val_hbm, idx_hbm, out_hbm, idx_buf, lid_buf, pos_buf,
          rows_buf, acc):
    c = lax.axis_index("c")
    s = lax.axis_index("s")

    pltpu.sync_copy(idx_hbm.at[pl.ds(s * SLICE, SLICE)], idx_buf)

    lanes = lax.iota(jnp.int32, 16)

    def one_pass(p, _):
        range_id = p * NC + c
        base = range_id * RANGE
        gbase = base + s * TROWS

        pltpu.sync_copy(mem_hbm.at[pl.ds(gbase, TROWS)],
                        acc.at[pl.ds(s * TROWS, TROWS)])

        def scan_chunk(i, cnt):
            idxv = idx_buf[pl.ds(i * 16, 16)]
            mi = 1 - jnp.minimum((idxv >> 14) ^ range_id, 1)
            pref = plsc.cumsum(mi)
            off = mi * (cnt + pref - mi) + (1 - mi) * DUMP
            plsc.store_scatter(lid_buf, [off], idxv - base)
            plsc.store_scatter(pos_buf, [off], s * SLICE + i * 16 + lanes)
            return cnt + jnp.sum(mi)

        cnt = lax.fori_loop(0, CHUNKS, scan_chunk, jnp.int32(0))

        plsc.subcore_barrier()

        def add_chunk(k, _):
            lidv = lid_buf[pl.ds(k * 16, 16)]
            posv = pos_buf[pl.ds(k * 16, 16)]
            t = k * 16 + lanes - cnt
            valid = (t >> 31) & 1
            lidv = valid * lidv + (1 - valid) * GARBAGE
            posv = valid * posv
            pltpu.sync_copy(val_hbm.at[posv], rows_buf)
            pltpu.sync_copy(rows_buf, acc.at[lidv], add=True)
            return 0

        nch = (cnt + 15) // 16
        lax.fori_loop(0, nch, add_chunk, 0)

        plsc.subcore_barrier()

        pltpu.sync_copy(acc.at[pl.ds(s * TROWS, TROWS)],
                        out_hbm.at[pl.ds(gbase, TROWS)])
        return 0

    lax.fori_loop(0, NPASS, one_pass, 0)


@jax.jit
def _scatter_add(mem, val, idx):
    mesh = plsc.VectorSubcoreMesh(core_axis_name="c", subcore_axis_name="s")
    return pl.kernel(
        _body,
        out_type=jax.ShapeDtypeStruct((M, D), jnp.float32),
        mesh=mesh,
        compiler_params=pltpu.CompilerParams(needs_layout_passes=False,
                                             use_tc_tiling_on_sc=False),
        scratch_types=[
            pltpu.VMEM((SLICE,), jnp.int32),
            pltpu.VMEM((SLICE + 16,), jnp.int32),
            pltpu.VMEM((SLICE + 16,), jnp.int32),
            pltpu.VMEM((16, D), jnp.float32),
            pltpu.VMEM_SHARED((RANGE + 8, D), jnp.float32),
        ],
    )(mem, val, idx)


def kernel(mem, val, idx):
    return _scatter_add(mem, val, idx)
